# Initial kernel scaffold; baseline (speedup 1.0000x reference)
import jax
import jax.numpy as jnp
from jax.experimental import pallas as pl


def kernel(emb_weight, learn_vectors, fixed_vectors, cooc, ids, ops, ix_func, ix_arg):
    raise NotImplementedError("write your pallas kernel here")


# speedup vs baseline: 92.2619x; 92.2619x over previous
import functools

import jax
import jax.numpy as jnp
from jax import lax
from jax.experimental import pallas as pl
from jax.experimental.pallas import tpu as pltpu
from jax.experimental.pallas import tpu_sc as plsc

DVEC = 64
SENT = 50
XP = 64
NC, NS, LANES = 2, 16, 16
NW = NC * NS


def _table_body(ids_ref, emb_ref, learn_ref, fixed_ref, cooc_ref, s_ref, x_ref):
    i = pl.program_id(0)

    @pl.when(i == 0)
    def _init():
        x_ref[...] = jnp.zeros_like(x_ref)

    row = jax.nn.softmax(emb_ref[0], axis=-1) * learn_ref[0, 0, 0] + fixed_ref[0]
    x_ref[pl.ds(i, 1), :] = row

    @pl.when(i == SENT - 1)
    def _finish():
        x = x_ref[...]
        for o in range(3):
            t = lax.dot_general(x, cooc_ref[o], (((1,), (0,)), ((), ())),
                                preferred_element_type=jnp.float32)
            s_ref[o] = lax.dot_general(t, x, (((1,), (1,)), ((), ())),
                                       preferred_element_type=jnp.float32)


def _score_table(emb_weight, learn_vectors, fixed_vectors, cooc, ids):
    grid_spec = pltpu.PrefetchScalarGridSpec(
        num_scalar_prefetch=1,
        grid=(SENT,),
        in_specs=[
            pl.BlockSpec((1, 1, DVEC), lambda i, ids: (ids[i], 0, 0)),
            pl.BlockSpec((1, 1, 1), lambda i, ids: (ids[i], 0, 0)),
            pl.BlockSpec((1, 1, DVEC), lambda i, ids: (ids[i], 0, 0)),
            pl.BlockSpec((3, DVEC, DVEC), lambda i, ids: (0, 0, 0)),
        ],
        out_specs=pl.BlockSpec((3, XP, XP), lambda i, ids: (0, 0, 0)),
        scratch_shapes=[pltpu.VMEM((XP, DVEC), jnp.float32)],
    )
    return pl.pallas_call(
        _table_body,
        grid_spec=grid_spec,
        out_shape=jax.ShapeDtypeStruct((3, XP, XP), jnp.float32),
    )(ids, emb_weight[:, None, :], learn_vectors[:, None, None], fixed_vectors[:, None, :], cooc)


def _make_sc_kernel(k, n1):
    per_w = k // NW
    ch = per_w // LANES
    mesh = plsc.VectorSubcoreMesh(core_axis_name="c", subcore_axis_name="s")

    @functools.partial(
        pl.kernel,
        out_type=jax.ShapeDtypeStruct((k,), jnp.float32),
        mesh=mesh,
        compiler_params=pltpu.CompilerParams(needs_layout_passes=False),
        scratch_types=[
            pltpu.VMEM((3 * XP * XP,), jnp.float32),
            pltpu.VMEM((ch * n1 * LANES,), jnp.int32),
            pltpu.VMEM((ch * n1 * LANES,), jnp.int32),
            pltpu.VMEM((ch * n1 * LANES,), jnp.int32),
            pltpu.VMEM((SENT * LANES,), jnp.int32),
            pltpu.VMEM((per_w,), jnp.float32),
        ],
    )
    def sc_kernel(s_hbm, ops_hbm, f_hbm, a_hbm, out_hbm,
                  s_v, ops_v, f_v, a_v, ptr_v, sc_v):
        w = lax.axis_index("s") * NC + lax.axis_index("c")
        pltpu.sync_copy(s_hbm, s_v)
        pltpu.sync_copy(ops_hbm.at[w], ops_v)
        pltpu.sync_copy(f_hbm.at[w], f_v)
        pltpu.sync_copy(a_hbm.at[w], a_v)
        lanes = lax.iota(jnp.int32, LANES)
        for j in range(ch):
            for s in range(SENT):
                plsc.store_scatter(ptr_v, [s * LANES + lanes],
                                   jnp.full((LANES,), s, jnp.int32))

            def step(i, acc, j=j):
                base = (j * n1 + i) * LANES + lanes
                opv = plsc.load_gather(ops_v, [base])
                fv = plsc.load_gather(f_v, [base])
                av = plsc.load_gather(a_v, [base])
                pf = plsc.load_gather(ptr_v, [fv * LANES + lanes])
                pa = plsc.load_gather(ptr_v, [av * LANES + lanes])
                val = plsc.load_gather(s_v, [opv * (XP * XP) + pf * XP + pa])
                plsc.store_scatter(ptr_v, [fv * LANES + lanes], pa, mask=opv == 2)
                return acc + val

            acc = lax.fori_loop(0, n1, step, jnp.zeros((LANES,), jnp.float32))
            plsc.store_scatter(sc_v, [j * LANES + lanes], acc)
        pltpu.sync_copy(sc_v, out_hbm.at[pl.ds(w * per_w, per_w)])

    return sc_kernel


def kernel(emb_weight, learn_vectors, fixed_vectors, cooc, ids, ops, ix_func, ix_arg):
    s_pad = _score_table(emb_weight, learn_vectors, fixed_vectors, cooc, ids)
    k, n1 = ops.shape
    sc_fn = _make_sc_kernel(k, n1)

    def pack(t):
        return t.reshape(NW, k // (NW * LANES), LANES, n1).transpose(0, 1, 3, 2).reshape(NW, -1)

    return sc_fn(s_pad.reshape(3 * XP * XP), pack(ops), pack(ix_func), pack(ix_arg))


# speedup vs baseline: 93.8310x; 1.0170x over previous
import functools

import jax
import jax.numpy as jnp
from jax import lax
from jax.experimental import pallas as pl
from jax.experimental.pallas import tpu as pltpu
from jax.experimental.pallas import tpu_sc as plsc

DVEC = 64
SENT = 50
XP = 64
NC, NS, LANES = 2, 16, 16
NW = NC * NS


def _table_body(ids_ref, emb_ref, learn_ref, fixed_ref, cooc_ref, s_ref, x_ref):
    i = pl.program_id(0)

    @pl.when(i == 0)
    def _init():
        x_ref[...] = jnp.zeros_like(x_ref)

    row = jax.nn.softmax(emb_ref[0], axis=-1) * learn_ref[0, 0, 0] + fixed_ref[0]
    x_ref[pl.ds(i, 1), :] = row

    @pl.when(i == SENT - 1)
    def _finish():
        x = x_ref[...]
        for o in range(3):
            t = lax.dot_general(x, cooc_ref[o], (((1,), (0,)), ((), ())),
                                preferred_element_type=jnp.float32)
            s_ref[o] = lax.dot_general(t, x, (((1,), (1,)), ((), ())),
                                       preferred_element_type=jnp.float32)


def _score_table(emb_weight, learn_vectors, fixed_vectors, cooc, ids):
    grid_spec = pltpu.PrefetchScalarGridSpec(
        num_scalar_prefetch=1,
        grid=(SENT,),
        in_specs=[
            pl.BlockSpec((1, 1, DVEC), lambda i, ids: (ids[i], 0, 0)),
            pl.BlockSpec((1, 1, 1), lambda i, ids: (ids[i], 0, 0)),
            pl.BlockSpec((1, 1, DVEC), lambda i, ids: (ids[i], 0, 0)),
            pl.BlockSpec((3, DVEC, DVEC), lambda i, ids: (0, 0, 0)),
        ],
        out_specs=pl.BlockSpec((3, XP, XP), lambda i, ids: (0, 0, 0)),
        scratch_shapes=[pltpu.VMEM((XP, DVEC), jnp.float32)],
    )
    return pl.pallas_call(
        _table_body,
        grid_spec=grid_spec,
        out_shape=jax.ShapeDtypeStruct((3, XP, XP), jnp.float32),
    )(ids, emb_weight[:, None, :], learn_vectors[:, None, None], fixed_vectors[:, None, :], cooc)


def _make_sc_kernel(k, n1):
    per_w = k // NW
    ch = per_w // LANES
    mesh = plsc.VectorSubcoreMesh(core_axis_name="c", subcore_axis_name="s")

    @functools.partial(
        pl.kernel,
        out_type=jax.ShapeDtypeStruct((k,), jnp.float32),
        mesh=mesh,
        compiler_params=pltpu.CompilerParams(needs_layout_passes=False),
        scratch_types=[
            pltpu.VMEM((3 * XP * XP,), jnp.float32),
            pltpu.VMEM((per_w * n1,), jnp.int32),
            pltpu.VMEM((per_w * n1,), jnp.int32),
            pltpu.VMEM((per_w * n1,), jnp.int32),
            pltpu.VMEM((SENT * LANES,), jnp.int32),
            pltpu.VMEM((per_w,), jnp.float32),
        ],
    )
    def sc_kernel(s_hbm, ops_hbm, f_hbm, a_hbm, out_hbm,
                  s_v, ops_v, f_v, a_v, ptr_v, sc_v):
        w = lax.axis_index("s") * NC + lax.axis_index("c")
        pltpu.sync_copy(s_hbm, s_v)
        pltpu.sync_copy(ops_hbm.at[pl.ds(w * per_w * n1, per_w * n1)], ops_v)
        pltpu.sync_copy(f_hbm.at[pl.ds(w * per_w * n1, per_w * n1)], f_v)
        pltpu.sync_copy(a_hbm.at[pl.ds(w * per_w * n1, per_w * n1)], a_v)
        lanes = lax.iota(jnp.int32, LANES)
        lanes_n1 = lanes * n1
        for j in range(ch):
            for s in range(SENT):
                plsc.store_scatter(ptr_v, [s * LANES + lanes],
                                   jnp.full((LANES,), s, jnp.int32))

            def step(i, acc, j=j):
                base = (j * LANES * n1 + i) + lanes_n1
                opv = plsc.load_gather(ops_v, [base])
                fv = plsc.load_gather(f_v, [base])
                av = plsc.load_gather(a_v, [base])
                pf = plsc.load_gather(ptr_v, [fv * LANES + lanes])
                pa = plsc.load_gather(ptr_v, [av * LANES + lanes])
                val = plsc.load_gather(s_v, [opv * (XP * XP) + pf * XP + pa])
                plsc.store_scatter(ptr_v, [fv * LANES + lanes], pa, mask=opv == 2)
                return acc + val

            acc = lax.fori_loop(0, n1, step, jnp.zeros((LANES,), jnp.float32))
            plsc.store_scatter(sc_v, [j * LANES + lanes], acc)
        pltpu.sync_copy(sc_v, out_hbm.at[pl.ds(w * per_w, per_w)])

    return sc_kernel


def kernel(emb_weight, learn_vectors, fixed_vectors, cooc, ids, ops, ix_func, ix_arg):
    s_pad = _score_table(emb_weight, learn_vectors, fixed_vectors, cooc, ids)
    k, n1 = ops.shape
    sc_fn = _make_sc_kernel(k, n1)
    return sc_fn(s_pad.reshape(3 * XP * XP), ops.reshape(-1),
                 ix_func.reshape(-1), ix_arg.reshape(-1))


# speedup vs baseline: 419.3120x; 4.4688x over previous
import functools

import jax
import jax.numpy as jnp
from jax import lax
from jax.experimental import pallas as pl
from jax.experimental.pallas import tpu as pltpu
from jax.experimental.pallas import tpu_sc as plsc

DVEC = 64
SENT = 50
XP = 64
NC, NS, LANES = 2, 16, 16
NW = NC * NS


def _table_body(emb_ref, learn_ref, fixed_ref, cooc_ref, s_ref, x_ref):
    x_ref[...] = jnp.zeros_like(x_ref)
    x = jax.nn.softmax(emb_ref[...], axis=-1) * learn_ref[...] + fixed_ref[...]
    x_ref[pl.ds(0, SENT), :] = x
    xp = x_ref[...]
    for o in range(3):
        t = lax.dot_general(xp, cooc_ref[o], (((1,), (0,)), ((), ())),
                            preferred_element_type=jnp.float32)
        s_ref[o] = lax.dot_general(t, xp, (((1,), (1,)), ((), ())),
                                   preferred_element_type=jnp.float32)


def _score_table(emb_rows, learn_rows, fixed_rows, cooc):
    return pl.pallas_call(
        _table_body,
        in_specs=[
            pl.BlockSpec((SENT, DVEC), lambda: (0, 0)),
            pl.BlockSpec((SENT, 1), lambda: (0, 0)),
            pl.BlockSpec((SENT, DVEC), lambda: (0, 0)),
            pl.BlockSpec((3, DVEC, DVEC), lambda: (0, 0, 0)),
        ],
        out_specs=pl.BlockSpec((3, XP, XP), lambda: (0, 0, 0)),
        out_shape=jax.ShapeDtypeStruct((3, XP, XP), jnp.float32),
        scratch_shapes=[pltpu.VMEM((XP, DVEC), jnp.float32)],
    )(emb_rows, learn_rows[:, None], fixed_rows, cooc)


def _make_sc_kernel(k, n1):
    per_w = k // NW
    ch = per_w // LANES
    mesh = plsc.VectorSubcoreMesh(core_axis_name="c", subcore_axis_name="s")

    @functools.partial(
        pl.kernel,
        out_type=jax.ShapeDtypeStruct((k,), jnp.float32),
        mesh=mesh,
        compiler_params=pltpu.CompilerParams(needs_layout_passes=False),
        scratch_types=[
            pltpu.VMEM((3 * XP * XP,), jnp.float32),
            pltpu.VMEM((per_w * n1,), jnp.int32),
            pltpu.VMEM((per_w * n1,), jnp.int32),
            pltpu.VMEM((per_w * n1,), jnp.int32),
            pltpu.VMEM((SENT * LANES,), jnp.int32),
            pltpu.VMEM((per_w,), jnp.float32),
        ],
    )
    def sc_kernel(s_hbm, ops_hbm, f_hbm, a_hbm, out_hbm,
                  s_v, ops_v, f_v, a_v, ptr_v, sc_v):
        w = lax.axis_index("s") * NC + lax.axis_index("c")
        pltpu.sync_copy(s_hbm, s_v)
        pltpu.sync_copy(ops_hbm.at[pl.ds(w * per_w * n1, per_w * n1)], ops_v)
        pltpu.sync_copy(f_hbm.at[pl.ds(w * per_w * n1, per_w * n1)], f_v)
        pltpu.sync_copy(a_hbm.at[pl.ds(w * per_w * n1, per_w * n1)], a_v)
        lanes = lax.iota(jnp.int32, LANES)
        lanes_n1 = lanes * n1
        for j in range(ch):
            for s in range(SENT):
                plsc.store_scatter(ptr_v, [s * LANES + lanes],
                                   jnp.full((LANES,), s, jnp.int32))

            def step(i, acc, j=j):
                base = (j * LANES * n1 + i) + lanes_n1
                opv = plsc.load_gather(ops_v, [base])
                fv = plsc.load_gather(f_v, [base])
                av = plsc.load_gather(a_v, [base])
                pf = plsc.load_gather(ptr_v, [fv * LANES + lanes])
                pa = plsc.load_gather(ptr_v, [av * LANES + lanes])
                val = plsc.load_gather(s_v, [opv * (XP * XP) + pf * XP + pa])
                plsc.store_scatter(ptr_v, [fv * LANES + lanes], pa, mask=opv == 2)
                return acc + val

            acc = lax.fori_loop(0, n1, step, jnp.zeros((LANES,), jnp.float32))
            plsc.store_scatter(sc_v, [j * LANES + lanes], acc)
        pltpu.sync_copy(sc_v, out_hbm.at[pl.ds(w * per_w, per_w)])

    return sc_kernel


def kernel(emb_weight, learn_vectors, fixed_vectors, cooc, ids, ops, ix_func, ix_arg):
    onehot = (ids[:, None] == jnp.arange(emb_weight.shape[0])[None, :]).astype(jnp.float32)
    emb_rows = onehot @ emb_weight
    fixed_rows = onehot @ fixed_vectors
    learn_rows = onehot @ learn_vectors
    s_pad = _score_table(emb_rows, learn_rows, fixed_rows, cooc)
    k, n1 = ops.shape
    sc_fn = _make_sc_kernel(k, n1)
    return sc_fn(s_pad.reshape(3 * XP * XP), ops.reshape(-1),
                 ix_func.reshape(-1), ix_arg.reshape(-1))
